# Initial kernel scaffold; baseline (speedup 1.0000x reference)
import jax
import jax.numpy as jnp
from jax.experimental import pallas as pl


def kernel(x):
    raise NotImplementedError("write your pallas kernel here")


# speedup vs baseline: 36.1611x; 36.1611x over previous
import jax
import jax.numpy as jnp
from jax import lax
from jax.experimental import pallas as pl
from jax.experimental.pallas import tpu as pltpu

P = 64
IMG = 512.0
CELL = IMG / P
PROB_T = 0.5
IOU_T = 0.5
N = P * P
ROWS = 32
LANES = 128


def _nms_body(x_ref, out_ref):
    shape = (ROWS, LANES)
    r = lax.broadcasted_iota(jnp.int32, shape, 0)
    c = lax.broadcasted_iota(jnp.int32, shape, 1)
    idx = r * LANES + c
    gx = (idx % P).astype(jnp.float32)
    gy = (idx // P).astype(jnp.float32)

    p = x_ref[0]
    b0 = x_ref[1]
    b1 = x_ref[2]
    b2 = x_ref[3]
    b3 = x_ref[4]

    x1 = (gx + b0) * CELL
    y1 = (gy + b1) * CELL
    x2 = x1 + jnp.abs(b2) * IMG
    y2 = y1 + jnp.abs(b3) * IMG
    area = (x2 - x1) * (y2 - y1)

    valid = p > PROB_T
    s = jnp.where(valid, p, 0.0)

    out_ref[...] = jnp.zeros_like(out_ref)

    lane = lax.broadcasted_iota(jnp.int32, (1, LANES), 1)

    def cand(st):
        supp, keep = st
        return valid & (supp == 0) & (keep == 0)

    def cond(st):
        return jnp.any(cand(st))

    def body(st):
        supp, keep = st
        m = cand(st)
        smax = jnp.max(jnp.where(m, s, -1.0))
        selm_s = m & (s == smax)
        isel = jnp.min(jnp.where(selm_s, idx, N))
        selmask = idx == isel
        keep = keep | selmask.astype(jnp.int32)
        xi = jnp.sum(jnp.where(selmask, x1, 0.0))
        yi = jnp.sum(jnp.where(selmask, y1, 0.0))
        Xi = jnp.sum(jnp.where(selmask, x2, 0.0))
        Yi = jnp.sum(jnp.where(selmask, y2, 0.0))
        ai = jnp.sum(jnp.where(selmask, area, 0.0))
        ix1 = jnp.maximum(xi, x1)
        iy1 = jnp.maximum(yi, y1)
        ix2 = jnp.minimum(Xi, x2)
        iy2 = jnp.minimum(Yi, y2)
        iw = jnp.maximum(ix2 - ix1, 0.0)
        ih = jnp.maximum(iy2 - iy1, 0.0)
        inter = iw * ih
        iou = inter / (ai + area - inter + 1e-9)
        later = (s < smax) | ((s == smax) & (idx > isel))
        supp = supp | ((iou > IOU_T) & later).astype(jnp.int32)
        n_gt = jnp.sum((s > smax).astype(jnp.int32))
        n_eq = jnp.sum(((s == smax) & (idx < isel)).astype(jnp.int32))
        rk = n_gt + n_eq
        row = jnp.where(
            lane == 0, smax,
            jnp.where(lane == 1, xi,
                      jnp.where(lane == 2, yi,
                                jnp.where(lane == 3, Xi,
                                          jnp.where(lane == 4, Yi, 0.0)))))
        out_ref[pl.ds(rk, 1), :] = row
        return supp, keep

    supp0 = jnp.zeros(shape, dtype=jnp.int32)
    keep0 = jnp.zeros(shape, dtype=jnp.int32)
    lax.while_loop(cond, body, (supp0, keep0))


def _nms_call(xr):
    return pl.pallas_call(
        _nms_body,
        out_shape=jax.ShapeDtypeStruct((N, LANES), jnp.float32),
    )(xr)


def kernel(x):
    xr = x.reshape(5, ROWS, LANES)
    out = _nms_call(xr)
    return out[:, :5]
